# baseline (speedup 1.0000x reference)
import functools

import jax
import jax.numpy as jnp
from jax import lax
from jax.experimental import pallas as pl
from jax.experimental.pallas import tpu as pltpu
from jax.experimental.pallas import tpu_sc as plsc

B = 16384
EMB = 32
NCOLS = 11
OTHERS = 128
NW = 32
BPW = B // NW
CHUNK = 128
NCHUNK = BPW // CHUNK


def _sc_gather(idx, tables):
    mesh = plsc.VectorSubcoreMesh(core_axis_name="c", subcore_axis_name="s")

    @functools.partial(
        pl.kernel,
        out_type=jax.ShapeDtypeStruct((NCOLS, B, EMB), jnp.float32),
        mesh=mesh,
        scratch_types=[
            pltpu.VMEM((NCOLS * BPW,), jnp.int32),
            pltpu.VMEM((BPW, EMB), jnp.float32),
            pltpu.VMEM((BPW, EMB), jnp.float32),
            pltpu.SemaphoreType.DMA,
            pltpu.SemaphoreType.DMA,
            pltpu.SemaphoreType.DMA,
        ],
        compiler_params=pltpu.CompilerParams(use_tc_tiling_on_sc=False),
    )
    def k(tab_hbm, idx_hbm, out_hbm,
          idx_v, buf0, buf1, gsem, wsem0, wsem1):
        wid = lax.axis_index("s") * 2 + lax.axis_index("c")
        base = wid * BPW
        idx_cps = [pltpu.async_copy(
            idx_hbm.at[pl.ds(c * B + base, BPW)],
            idx_v.at[pl.ds(c * BPW, BPW)], gsem) for c in range(NCOLS)]
        for cp in idx_cps:
            cp.wait()
        bufs = [buf0, buf1]
        wsems = [wsem0, wsem1]
        pending = [None, None]
        for c in range(NCOLS):
            p = c % 2
            buf = bufs[p]
            if pending[p] is not None:
                pending[p].wait()
            gathers = []
            for j in range(NCHUNK):
                gathers.append(pltpu.async_copy(
                    tab_hbm.at[idx_v.at[pl.ds(c * BPW + j * CHUNK, CHUNK)]],
                    buf.at[pl.ds(j * CHUNK, CHUNK)],
                    gsem))
            for cp in gathers:
                cp.wait()
            pending[p] = pltpu.async_copy(
                buf, out_hbm.at[c, pl.ds(base, BPW)], wsems[p])
        for p in range(2):
            if pending[p] is not None:
                pending[p].wait()

    return k(tables, idx)


def _tc_head(emb, others, w2, b2):
    bm = 2048

    def body(emb_ref, oth_ref, w2_ref, b_ref, out_ref):
        acc = jnp.dot(oth_ref[...], w2_ref[NCOLS * EMB:, :],
                      preferred_element_type=jnp.float32)
        for c in range(NCOLS):
            acc = acc + jnp.dot(emb_ref[c], w2_ref[c * EMB:(c + 1) * EMB, :],
                                preferred_element_type=jnp.float32)
        out_ref[...] = jnp.maximum(acc + b_ref[0], 0.0)

    return pl.pallas_call(
        body,
        grid=(B // bm,),
        in_specs=[
            pl.BlockSpec((NCOLS, bm, EMB), lambda i: (0, i, 0)),
            pl.BlockSpec((bm, OTHERS), lambda i: (i, 0)),
            pl.BlockSpec((NCOLS * EMB + OTHERS, EMB), lambda i: (0, 0)),
            pl.BlockSpec((1, EMB), lambda i: (0, 0)),
        ],
        out_specs=pl.BlockSpec((bm, EMB), lambda i: (i, 0)),
        out_shape=jax.ShapeDtypeStruct((B, EMB), jnp.float32),
    )(emb, others, w2, b2)


def kernel(state, species_table, item_table, ability_table, move_table, W, b):
    V = species_table.shape[0]
    tables = jnp.concatenate(
        [species_table, item_table, ability_table, move_table], axis=0)
    offs = jnp.array([0, V, 2 * V, 2 * V, 2 * V, 2 * V, 2 * V,
                      3 * V, 3 * V, 3 * V, 3 * V], dtype=jnp.int32)
    idx = (state[:, :NCOLS].astype(jnp.int32) + offs[None, :]).T.reshape(-1)
    others = state[:, NCOLS:]
    Wt = W.T
    w2 = jnp.concatenate([
        Wt[0:3 * EMB],
        jnp.tile(Wt[3 * EMB:4 * EMB] * 0.25, (4, 1)),
        jnp.tile(Wt[4 * EMB:5 * EMB] * 0.25, (4, 1)),
        Wt[5 * EMB:],
    ], axis=0)
    emb = _sc_gather(idx, tables)
    return _tc_head(emb, others, w2, b.reshape(1, EMB))


# speedup vs baseline: 1.5274x; 1.5274x over previous
import functools

import jax
import jax.numpy as jnp
from jax import lax
from jax.experimental import pallas as pl
from jax.experimental.pallas import tpu as pltpu
from jax.experimental.pallas import tpu_sc as plsc

B = 16384
EMB = 32
NCOLS = 11
OTHERS = 128
NW = 32
BPW = B // NW
CHUNK = 128
NCHUNK = BPW // CHUNK


def _sc_gather(idx, species, item, ability, move):
    mesh = plsc.VectorSubcoreMesh(core_axis_name="c", subcore_axis_name="s")

    @functools.partial(
        pl.kernel,
        out_type=jax.ShapeDtypeStruct((NCOLS, B, EMB), jnp.float32),
        mesh=mesh,
        scratch_types=[
            pltpu.VMEM((BPW * NCOLS,), jnp.int32),
            pltpu.VMEM((NCOLS * BPW,), jnp.int32),
            pltpu.VMEM((BPW, EMB), jnp.float32),
            pltpu.VMEM((BPW, EMB), jnp.float32),
            pltpu.SemaphoreType.DMA,
            pltpu.SemaphoreType.DMA,
            pltpu.SemaphoreType.DMA,
        ],
        compiler_params=pltpu.CompilerParams(
            use_tc_tiling_on_sc=False, needs_layout_passes=False),
    )
    def k(sp_hbm, it_hbm, ab_hbm, mv_hbm, idx_hbm, out_hbm,
          raw_v, idx_v, buf0, buf1, gsem, wsem0, wsem1):
        wid = lax.axis_index("s") * 2 + lax.axis_index("c")
        base = wid * BPW
        pltpu.sync_copy(idx_hbm.at[pl.ds(base * NCOLS, BPW * NCOLS)], raw_v)
        lane = lax.iota(jnp.int32, 16) * NCOLS
        for c in range(NCOLS):

            @pl.loop(0, BPW, step=16)
            def _(r, c=c):
                vals = plsc.load_gather(raw_v, [r * NCOLS + c + lane])
                idx_v[pl.ds(c * BPW + r, 16)] = vals

        tables = [sp_hbm, it_hbm, ab_hbm, ab_hbm, ab_hbm, ab_hbm, ab_hbm,
                  mv_hbm, mv_hbm, mv_hbm, mv_hbm]
        bufs = [buf0, buf1]
        wsems = [wsem0, wsem1]
        pending = [None, None]
        for c in range(NCOLS):
            p = c % 2
            buf = bufs[p]
            if pending[p] is not None:
                pending[p].wait()
            gathers = []
            for j in range(NCHUNK):
                gathers.append(pltpu.async_copy(
                    tables[c].at[idx_v.at[pl.ds(c * BPW + j * CHUNK, CHUNK)]],
                    buf.at[pl.ds(j * CHUNK, CHUNK)],
                    gsem))
            for cp in gathers:
                cp.wait()
            pending[p] = pltpu.async_copy(
                buf, out_hbm.at[c, pl.ds(base, BPW)], wsems[p])
        for p in range(2):
            if pending[p] is not None:
                pending[p].wait()

    return k(species, item, ability, move, idx)


def _tc_head(emb, others, w2, b2):
    bm = 2048

    def body(emb_ref, oth_ref, w2_ref, b_ref, out_ref):
        acc = jnp.dot(oth_ref[...], w2_ref[NCOLS * EMB:, :],
                      preferred_element_type=jnp.float32)
        for c in range(NCOLS):
            acc = acc + jnp.dot(emb_ref[c], w2_ref[c * EMB:(c + 1) * EMB, :],
                                preferred_element_type=jnp.float32)
        out_ref[...] = jnp.maximum(acc + b_ref[0], 0.0)

    return pl.pallas_call(
        body,
        grid=(B // bm,),
        in_specs=[
            pl.BlockSpec((NCOLS, bm, EMB), lambda i: (0, i, 0)),
            pl.BlockSpec((bm, OTHERS), lambda i: (i, 0)),
            pl.BlockSpec((NCOLS * EMB + OTHERS, EMB), lambda i: (0, 0)),
            pl.BlockSpec((1, EMB), lambda i: (0, 0)),
        ],
        out_specs=pl.BlockSpec((bm, EMB), lambda i: (i, 0)),
        out_shape=jax.ShapeDtypeStruct((B, EMB), jnp.float32),
    )(emb, others, w2, b2)


def kernel(state, species_table, item_table, ability_table, move_table, W, b):
    idx = state[:, :NCOLS].astype(jnp.int32).reshape(-1)
    others = state[:, NCOLS:]
    Wt = W.T
    w2 = jnp.concatenate([
        Wt[0:3 * EMB],
        jnp.tile(Wt[3 * EMB:4 * EMB] * 0.25, (4, 1)),
        jnp.tile(Wt[4 * EMB:5 * EMB] * 0.25, (4, 1)),
        Wt[5 * EMB:],
    ], axis=0)
    emb = _sc_gather(idx, species_table, item_table, ability_table, move_table)
    return _tc_head(emb, others, w2, b.reshape(1, EMB))


# speedup vs baseline: 1.9860x; 1.3003x over previous
import functools

import jax
import jax.numpy as jnp
from jax import lax
from jax.experimental import pallas as pl
from jax.experimental.pallas import tpu as pltpu
from jax.experimental.pallas import tpu_sc as plsc

B = 16384
VOCAB = 100000
EMB = 32
NCOLS = 11
OTHERS = 128
NW = 32
BPW = B // NW
CHUNK = 128
NCHUNK = BPW // CHUNK
VROWS = NCOLS * B * EMB // 128
RPP = B * EMB // 128


def _sc_gather(idx, species, item, ability, move):
    mesh = plsc.VectorSubcoreMesh(core_axis_name="c", subcore_axis_name="s")

    @functools.partial(
        pl.kernel,
        out_type=jax.ShapeDtypeStruct((NCOLS * B, EMB), jnp.float32),
        mesh=mesh,
        scratch_types=[
            pltpu.VMEM((NCOLS * BPW,), jnp.int32),
            pltpu.VMEM((BPW, EMB), jnp.float32),
            pltpu.VMEM((BPW, EMB), jnp.float32),
            pltpu.SemaphoreType.DMA,
            pltpu.SemaphoreType.DMA,
            pltpu.SemaphoreType.DMA,
        ],
        compiler_params=pltpu.CompilerParams(use_tc_tiling_on_sc=False),
    )
    def k(sp_hbm, it_hbm, ab_hbm, mv_hbm, idx_hbm, out_hbm,
          idx_v, buf0, buf1, gsem, wsem0, wsem1):
        wid = lax.axis_index("s") * 2 + lax.axis_index("c")
        base = wid * BPW
        idx_cps = [pltpu.async_copy(
            idx_hbm.at[pl.ds(c * B + base, BPW)],
            idx_v.at[pl.ds(c * BPW, BPW)], gsem) for c in range(NCOLS)]
        for cp in idx_cps:
            cp.wait()
        tables = [sp_hbm, it_hbm, ab_hbm, ab_hbm, ab_hbm, ab_hbm, ab_hbm,
                  mv_hbm, mv_hbm, mv_hbm, mv_hbm]
        bufs = [buf0, buf1]
        wsems = [wsem0, wsem1]
        pending = [None, None]
        for c in range(NCOLS):
            p = c % 2
            buf = bufs[p]
            if pending[p] is not None:
                pending[p].wait()
            gathers = []
            for j in range(NCHUNK):
                gathers.append(pltpu.async_copy(
                    tables[c].at[idx_v.at[pl.ds(c * BPW + j * CHUNK, CHUNK)]],
                    buf.at[pl.ds(j * CHUNK, CHUNK)],
                    gsem))
            for cp in gathers:
                cp.wait()
            pending[p] = pltpu.async_copy(
                buf, out_hbm.at[pl.ds(c * B + base, BPW)], wsems[p])
        for p in range(2):
            if pending[p] is not None:
                pending[p].wait()

    return k(species, item, ability, move, idx)


def _tc_head(emb_view, others, bd, wo, b4):
    bm = 2048
    bv = bm // 4

    def body(*refs):
        emb_refs = refs[:NCOLS]
        oth_ref, bd_ref, wo_ref, b4_ref, out_ref = refs[NCOLS:]
        acc = jnp.dot(emb_refs[0][...], bd_ref[0],
                      preferred_element_type=jnp.float32)
        for c in range(1, NCOLS):
            acc = acc + jnp.dot(emb_refs[c][...], bd_ref[c],
                                preferred_element_type=jnp.float32)
        acc = acc + jnp.dot(oth_ref[...], wo_ref[...],
                            preferred_element_type=jnp.float32)
        out_ref[...] = jnp.maximum(acc + b4_ref[0], 0.0)

    emb_specs = [
        pl.BlockSpec((bv, 128), functools.partial(
            lambda i, c: (c * (RPP // bv) + i, 0), c=c))
        for c in range(NCOLS)
    ]
    return pl.pallas_call(
        body,
        grid=(B // bm,),
        in_specs=emb_specs + [
            pl.BlockSpec((bv, 4 * OTHERS), lambda i: (i, 0)),
            pl.BlockSpec((NCOLS, 128, 128), lambda i: (0, 0, 0)),
            pl.BlockSpec((4 * OTHERS, 128), lambda i: (0, 0)),
            pl.BlockSpec((1, 128), lambda i: (0, 0)),
        ],
        out_specs=pl.BlockSpec((bv, 128), lambda i: (i, 0)),
        out_shape=jax.ShapeDtypeStruct((B * EMB // 128, 128), jnp.float32),
    )(*([emb_view] * NCOLS), others, bd, wo, b4)


def kernel(state, species_table, item_table, ability_table, move_table, W, b):
    idx = state[:, :NCOLS].astype(jnp.int32).T.reshape(-1)
    others = state[:, NCOLS:].reshape(B // 4, 4 * OTHERS)
    Wt = W.T
    eye4 = jnp.eye(4, dtype=jnp.float32)
    scales = [1.0, 1.0, 1.0] + [0.25] * 8
    bd = jnp.stack([
        jnp.kron(eye4, Wt[c * EMB:(c + 1) * EMB] * scales[c])
        for c in range(3)] + [
        jnp.kron(eye4, Wt[3 * EMB:4 * EMB] * 0.25) for _ in range(4)] + [
        jnp.kron(eye4, Wt[4 * EMB:5 * EMB] * 0.25) for _ in range(4)])
    wo = jnp.kron(eye4, Wt[5 * EMB:])
    b4 = jnp.tile(b, 4).reshape(1, 128)
    emb = _sc_gather(idx, species_table, item_table,
                     ability_table, move_table)
    emb_view = emb.reshape(VROWS, 128)
    out_view = _tc_head(emb_view, others, bd, wo, b4)
    return out_view.reshape(B, EMB)


# speedup vs baseline: 2.0515x; 1.0330x over previous
import functools

import jax
import jax.numpy as jnp
from jax import lax
from jax.experimental import pallas as pl
from jax.experimental.pallas import tpu as pltpu
from jax.experimental.pallas import tpu_sc as plsc

B = 16384
VOCAB = 100000
EMB = 32
NCOLS = 11
OTHERS = 128
NW = 32
BPW = B // NW
CHUNK = 128
NCHUNK = BPW // CHUNK
VROWS = NCOLS * B * EMB // 128
RPP = B * EMB // 128


def _sc_gather(idx, species, item, ability, move):
    mesh = plsc.VectorSubcoreMesh(core_axis_name="c", subcore_axis_name="s")

    @functools.partial(
        pl.kernel,
        out_type=jax.ShapeDtypeStruct((NCOLS * B, EMB), jnp.float32),
        mesh=mesh,
        scratch_types=[
            pltpu.VMEM((NCOLS * BPW,), jnp.int32),
            pltpu.VMEM((BPW, EMB), jnp.float32),
            pltpu.VMEM((BPW, EMB), jnp.float32),
            pltpu.SemaphoreType.DMA,
            pltpu.SemaphoreType.DMA,
            pltpu.SemaphoreType.DMA,
        ],
        compiler_params=pltpu.CompilerParams(use_tc_tiling_on_sc=False),
    )
    def k(sp_hbm, it_hbm, ab_hbm, mv_hbm, idx_hbm, out_hbm,
          idx_v, buf0, buf1, gsem, wsem0, wsem1):
        wid = lax.axis_index("s") * 2 + lax.axis_index("c")
        base = wid * BPW
        idx_cps = [pltpu.async_copy(
            idx_hbm.at[pl.ds(c * B + base, BPW)],
            idx_v.at[pl.ds(c * BPW, BPW)], gsem) for c in range(NCOLS)]
        for cp in idx_cps:
            cp.wait()
        tables = [sp_hbm, it_hbm, ab_hbm, ab_hbm, ab_hbm, ab_hbm, ab_hbm,
                  mv_hbm, mv_hbm, mv_hbm, mv_hbm]
        bufs = [buf0, buf1]
        wsems = [wsem0, wsem1]
        pending = [None, None]
        for c in range(NCOLS):
            p = c % 2
            buf = bufs[p]
            if pending[p] is not None:
                pending[p].wait()
            gathers = []
            for j in range(NCHUNK):
                gathers.append(pltpu.async_copy(
                    tables[c].at[idx_v.at[pl.ds(c * BPW + j * CHUNK, CHUNK)]],
                    buf.at[pl.ds(j * CHUNK, CHUNK)],
                    gsem))
            for cp in gathers:
                cp.wait()
            pending[p] = pltpu.async_copy(
                buf, out_hbm.at[pl.ds(c * B + base, BPW)], wsems[p])
        for p in range(2):
            if pending[p] is not None:
                pending[p].wait()

    return k(species, item, ability, move, idx)


def _tc_untile(tab_t):
    bc = 4096

    def body(x_ref, o_ref):
        t = jnp.transpose(x_ref[...])
        t3 = t.reshape(bc // 4, 4, EMB)
        for g in range(4):
            o_ref[:, g * EMB:(g + 1) * EMB] = t3[:, g, :]

    return pl.pallas_call(
        body,
        grid=(pl.cdiv(VOCAB, bc),),
        in_specs=[pl.BlockSpec((EMB, bc), lambda i: (0, i))],
        out_specs=pl.BlockSpec((bc // 4, 128), lambda i: (i, 0)),
        out_shape=jax.ShapeDtypeStruct((VOCAB // 4, 128), jnp.float32),
    )(tab_t)


def _tc_head(emb_view, others, bd, wo, b4):
    bm = 2048
    bv = bm // 4

    def body(*refs):
        emb_refs = refs[:NCOLS]
        oth_ref, bd_ref, wo_ref, b4_ref, out_ref = refs[NCOLS:]
        acc = jnp.dot(emb_refs[0][...], bd_ref[0],
                      preferred_element_type=jnp.float32)
        for c in range(1, NCOLS):
            acc = acc + jnp.dot(emb_refs[c][...], bd_ref[c],
                                preferred_element_type=jnp.float32)
        acc = acc + jnp.dot(oth_ref[...], wo_ref[...],
                            preferred_element_type=jnp.float32)
        out_ref[...] = jnp.maximum(acc + b4_ref[0], 0.0)

    emb_specs = [
        pl.BlockSpec((bv, 128), functools.partial(
            lambda i, c: (c * (RPP // bv) + i, 0), c=c))
        for c in range(NCOLS)
    ]
    return pl.pallas_call(
        body,
        grid=(B // bm,),
        in_specs=emb_specs + [
            pl.BlockSpec((bv, 4 * OTHERS), lambda i: (i, 0)),
            pl.BlockSpec((NCOLS, 128, 128), lambda i: (0, 0, 0)),
            pl.BlockSpec((4 * OTHERS, 128), lambda i: (0, 0)),
            pl.BlockSpec((1, 128), lambda i: (0, 0)),
        ],
        out_specs=pl.BlockSpec((bv, 128), lambda i: (i, 0)),
        out_shape=jax.ShapeDtypeStruct((B * EMB // 128, 128), jnp.float32),
    )(*([emb_view] * NCOLS), others, bd, wo, b4)


def kernel(state, species_table, item_table, ability_table, move_table, W, b):
    idx = state[:, :NCOLS].astype(jnp.int32).T.reshape(-1)
    others = state[:, NCOLS:].reshape(B // 4, 4 * OTHERS)
    Wt = W.T
    eye4 = jnp.eye(4, dtype=jnp.float32)
    scales = [1.0, 1.0, 1.0] + [0.25] * 8
    bd = jnp.stack([
        jnp.kron(eye4, Wt[c * EMB:(c + 1) * EMB] * scales[c])
        for c in range(3)] + [
        jnp.kron(eye4, Wt[3 * EMB:4 * EMB] * 0.25) for _ in range(4)] + [
        jnp.kron(eye4, Wt[4 * EMB:5 * EMB] * 0.25) for _ in range(4)])
    wo = jnp.kron(eye4, Wt[5 * EMB:])
    b4 = jnp.tile(b, 4).reshape(1, 128)
    tabs = [_tc_untile(t.T).reshape(VOCAB, EMB) for t in
            (species_table, item_table, ability_table, move_table)]
    emb = _sc_gather(idx, *tabs)
    emb_view = emb.reshape(VROWS, 128)
    out_view = _tc_head(emb_view, others, bd, wo, b4)
    return out_view.reshape(B, EMB)


# speedup vs baseline: 2.1273x; 1.0369x over previous
import functools

import jax
import jax.numpy as jnp
from jax import lax
from jax.experimental import pallas as pl
from jax.experimental.pallas import tpu as pltpu
from jax.experimental.pallas import tpu_sc as plsc

B = 16384
VOCAB = 100000
EMB = 32
NCOLS = 11
OTHERS = 128
NW = 32
BPW = B // NW
CHUNK = 128
NCHUNK = BPW // CHUNK
VROWS = NCOLS * B * EMB // 128
RPP = B * EMB // 128


def _sc_gather(idx, table, cols):
    nc = len(cols)
    mesh = plsc.VectorSubcoreMesh(core_axis_name="c", subcore_axis_name="s")

    @functools.partial(
        pl.kernel,
        out_type=jax.ShapeDtypeStruct((nc * B, EMB), jnp.float32),
        mesh=mesh,
        scratch_types=[
            pltpu.VMEM((nc * BPW,), jnp.int32),
            pltpu.VMEM((BPW, EMB), jnp.float32),
            pltpu.VMEM((BPW, EMB), jnp.float32),
            pltpu.SemaphoreType.DMA,
            pltpu.SemaphoreType.DMA,
            pltpu.SemaphoreType.DMA,
        ],
        compiler_params=pltpu.CompilerParams(use_tc_tiling_on_sc=False),
    )
    def k(tab_hbm, idx_hbm, out_hbm, idx_v, buf0, buf1, gsem, wsem0, wsem1):
        wid = lax.axis_index("s") * 2 + lax.axis_index("c")
        base = wid * BPW
        idx_cps = [pltpu.async_copy(
            idx_hbm.at[pl.ds(c * B + base, BPW)],
            idx_v.at[pl.ds(p * BPW, BPW)], gsem)
            for p, c in enumerate(cols)]
        for cp in idx_cps:
            cp.wait()
        bufs = [buf0, buf1]
        wsems = [wsem0, wsem1]
        pending = [None, None]
        for p in range(nc):
            par = p % 2
            buf = bufs[par]
            if pending[par] is not None:
                pending[par].wait()
            gathers = []
            for j in range(NCHUNK):
                gathers.append(pltpu.async_copy(
                    tab_hbm.at[idx_v.at[pl.ds(p * BPW + j * CHUNK, CHUNK)]],
                    buf.at[pl.ds(j * CHUNK, CHUNK)],
                    gsem))
            for cp in gathers:
                cp.wait()
            pending[par] = pltpu.async_copy(
                buf, out_hbm.at[pl.ds(p * B + base, BPW)], wsems[par])
        for par in range(2):
            if pending[par] is not None:
                pending[par].wait()

    return k(table, idx)


def _tc_untile(tab_t):
    bc = 4096

    def body(x_ref, o_ref):
        t = jnp.transpose(x_ref[...])
        t3 = t.reshape(bc // 4, 4, EMB)
        for g in range(4):
            o_ref[:, g * EMB:(g + 1) * EMB] = t3[:, g, :]

    return pl.pallas_call(
        body,
        grid=(pl.cdiv(VOCAB, bc),),
        in_specs=[pl.BlockSpec((EMB, bc), lambda i: (0, i))],
        out_specs=pl.BlockSpec((bc // 4, 128), lambda i: (i, 0)),
        out_shape=jax.ShapeDtypeStruct((VOCAB // 4, 128), jnp.float32),
    )(tab_t)


def _tc_head(emb_view, others, bd, wo, b4):
    bm = 2048
    bv = bm // 4

    def body(*refs):
        emb_refs = refs[:NCOLS]
        oth_ref, bd_ref, wo_ref, b4_ref, out_ref = refs[NCOLS:]
        acc = jnp.dot(emb_refs[0][...], bd_ref[0],
                      preferred_element_type=jnp.float32)
        for c in range(1, NCOLS):
            acc = acc + jnp.dot(emb_refs[c][...], bd_ref[c],
                                preferred_element_type=jnp.float32)
        acc = acc + jnp.dot(oth_ref[...], wo_ref[...],
                            preferred_element_type=jnp.float32)
        out_ref[...] = jnp.maximum(acc + b4_ref[0], 0.0)

    views, locals_ = zip(*emb_view)
    emb_specs = [
        pl.BlockSpec((bv, 128), functools.partial(
            lambda i, p: (p * (RPP // bv) + i, 0), p=p))
        for p in locals_
    ]
    return pl.pallas_call(
        body,
        grid=(B // bm,),
        in_specs=emb_specs + [
            pl.BlockSpec((bv, 4 * OTHERS), lambda i: (i, 0)),
            pl.BlockSpec((NCOLS, 128, 128), lambda i: (0, 0, 0)),
            pl.BlockSpec((4 * OTHERS, 128), lambda i: (0, 0)),
            pl.BlockSpec((1, 128), lambda i: (0, 0)),
        ],
        out_specs=pl.BlockSpec((bv, 128), lambda i: (i, 0)),
        out_shape=jax.ShapeDtypeStruct((B * EMB // 128, 128), jnp.float32),
    )(*views, others, bd, wo, b4)


def kernel(state, species_table, item_table, ability_table, move_table, W, b):
    idx = state[:, :NCOLS].astype(jnp.int32).T.reshape(-1)
    others = state[:, NCOLS:].reshape(B // 4, 4 * OTHERS)
    Wt = W.T
    eye4 = jnp.eye(4, dtype=jnp.float32)
    scales = [1.0, 1.0, 1.0] + [0.25] * 8
    bd = jnp.stack([
        jnp.kron(eye4, Wt[c * EMB:(c + 1) * EMB] * scales[c])
        for c in range(3)] + [
        jnp.kron(eye4, Wt[3 * EMB:4 * EMB] * 0.25) for _ in range(4)] + [
        jnp.kron(eye4, Wt[4 * EMB:5 * EMB] * 0.25) for _ in range(4)])
    wo = jnp.kron(eye4, Wt[5 * EMB:])
    b4 = jnp.tile(b, 4).reshape(1, 128)
    tab_ab = _tc_untile(ability_table.T).reshape(VOCAB, EMB)
    tab_mv = _tc_untile(move_table.T).reshape(VOCAB, EMB)
    tab_sp = _tc_untile(species_table.T).reshape(VOCAB, EMB)
    tab_it = _tc_untile(item_table.T).reshape(VOCAB, EMB)
    emb_ab = _sc_gather(idx, tab_ab, [2, 3, 4, 5, 6]).reshape(-1, 128)
    emb_mv = _sc_gather(idx, tab_mv, [7, 8, 9, 10]).reshape(-1, 128)
    emb_sp = _sc_gather(idx, tab_sp, [0]).reshape(-1, 128)
    emb_it = _sc_gather(idx, tab_it, [1]).reshape(-1, 128)
    plane_views = ([(emb_sp, 0), (emb_it, 0)]
                   + [(emb_ab, p) for p in range(5)]
                   + [(emb_mv, p) for p in range(4)])
    out_view = _tc_head(plane_views, others, bd, wo, b4)
    return out_view.reshape(B, EMB)
